# baseline (speedup 1.0000x reference)
import functools

import jax
import jax.numpy as jnp
from jax import lax
from jax.experimental import pallas as pl
from jax.experimental.pallas import tpu as pltpu
from jax.experimental.pallas import tpu_sc as plsc

_CUTS = (0, 50000, 100000, 180000, 200000)
_SIZES = (50000, 50000, 80000, 20000)
_D_PROJ = 512
_SCALE = float(_D_PROJ) ** 0.5

_GDIM = (512, 128, 128, 128)
_PACK = (1, 1, 4, 16)
_SHIFTS = (0, 0, 2, 4)

_NC, _NS = 2, 16
_NW = _NC * _NS
_TOK = 4 * 8192
_TPW = _TOK // _NW
_CH = 64
_CHB = 6
_NCH = _TPW // _CH
_GROWS = _TOK + _CH

_BT = 1024


def _sc_gather(inp_flat, t0, t1, t2p, t3p):
    mesh = plsc.VectorSubcoreMesh(core_axis_name="c", subcore_axis_name="s")
    out_type = tuple(
        jax.ShapeDtypeStruct((_GROWS, d), jnp.float32) for d in _GDIM
    )
    scratch = [pltpu.VMEM((_TPW,), jnp.int32)]
    scratch += [pltpu.VMEM((_TPW + 16,), jnp.int32)] * 4
    scratch += [pltpu.VMEM((_TPW + 16,), jnp.int32)] * 4
    scratch += [pltpu.VMEM((_NCH, _CH), jnp.int32)] * 4
    scratch += [pltpu.VMEM((_CH, _GDIM[0]), jnp.float32)] * 2
    for d in _GDIM[1:3]:
        scratch += [pltpu.VMEM((_CH, d), jnp.float32)] * 2
    scratch += [pltpu.VMEM((_CH, _GDIM[3]), jnp.float32)]
    scratch += [pltpu.SemaphoreType.DMA] * 16
    scratch += [pltpu.VMEM((48,), jnp.int32)]

    @functools.partial(
        pl.kernel,
        out_type=out_type,
        mesh=mesh,
        scratch_types=scratch,
    )
    def body(inp_hbm, t0h, t1h, t2h, t3h, g0, g1, g2, g3,
             tok_v, *rest):
        cidx = rest[0:4]
        ctok = rest[4:8]
        cdst = rest[8:12]
        rows = [(rest[12], rest[13]), (rest[14], rest[15]),
                (rest[16], rest[17]), (rest[18], rest[18])]
        gsem = [(rest[19 + 2 * b], rest[20 + 2 * b]) for b in range(4)]
        ssem = [(rest[27 + 2 * b], rest[28 + 2 * b]) for b in range(4)]
        psum = rest[35]
        wid = lax.axis_index("s") * _NC + lax.axis_index("c")
        base = wid * _TPW
        pltpu.sync_copy(inp_hbm.at[pl.ds(base, _TPW)], tok_v)

        zero16 = jnp.zeros((16,), jnp.int32)
        trash16 = zero16 + _TOK
        psum[pl.ds(0, 16)] = zero16

        def prefill(j, carry):
            for b in range(4):
                cidx[b][pl.ds(j * 16, 16)] = zero16
                ctok[b][pl.ds(j * 16, 16)] = trash16
            return carry

        lax.fori_loop(0, (_TPW + 16) // 16, prefill, 0)

        def prefix16(v):
            for sh in (1, 2, 4, 8):
                psum[pl.ds(16, 16)] = v
                v = v + psum[pl.ds(16 - sh, 16)]
            return v

        lane = lax.iota(jnp.int32, 16)
        lane1p = lane + 1

        def compact(j, counts):
            x = tok_v[pl.ds(j * 16, 16)]
            tokpos = base + j * 16 + lane
            new_counts = []
            for b in range(4):
                m = (x >= _CUTS[b]) & (x < _CUTS[b + 1])
                ridx = jnp.minimum(
                    jnp.maximum(x - _CUTS[b], 0), _SIZES[b] - 1)
                if _SHIFTS[b]:
                    ridx = lax.shift_right_logical(ridx, _SHIFTS[b])
                cnt = counts[b]
                mi = jnp.where(m, 1, 0)
                pc = prefix16(mi)
                cv1 = zero16
                cv2 = trash16
                for k in range(16):
                    cond = lane1p == pc[k] * mi[k]
                    cv1 = jnp.where(cond, ridx[k], cv1)
                    cv2 = jnp.where(cond, tokpos[k], cv2)
                cidx[b][pl.ds(cnt, 16)] = cv1
                ctok[b][pl.ds(cnt, 16)] = cv2
                new_counts.append(cnt + pc[15])
            return tuple(new_counts)

        counts = lax.fori_loop(
            0, _TPW // 16, compact,
            (jnp.int32(0), jnp.int32(0), jnp.int32(0), jnp.int32(0)))

        def layout_dst(j, carry):
            for b in range(4):
                cdst[b][j // 4, pl.ds((j % 4) * 16, 16)] = (
                    ctok[b][pl.ds(j * 16, 16)])
            return carry

        for j in range(_TPW // 16):
            layout_dst(j, 0)

        tbls = (t0h, t1h, t2h, t3h)
        gouts = (g0, g1, g2, g3)
        nbufs = (2, 2, 2, 1)
        for b in range(4):
            tbl, g, nb = tbls[b], gouts[b], nbufs[b]
            nch = lax.shift_right_logical(counts[b] + (_CH - 1), _CHB)
            for c in range(_NCH):
                @pl.when(c < nch)
                def _(b=b, c=c, tbl=tbl, g=g, nb=nb):
                    buf = rows[b][c % nb]
                    if c >= nb:
                        pltpu.make_async_copy(
                            buf, g.at[cdst[b].at[c - nb]],
                            ssem[b][c % nb]).wait()
                    pltpu.async_copy(
                        tbl.at[cidx[b].at[pl.ds(c * _CH, _CH)]],
                        buf, gsem[b][c % nb]).wait()
                    pltpu.async_copy(
                        buf, g.at[cdst[b].at[c]], ssem[b][c % nb])
            for c in range(_NCH):
                @pl.when(jnp.logical_and(c < nch, c + nbufs[b] >= nch))
                def _(b=b, c=c, g=g, nb=nb):
                    pltpu.make_async_copy(
                        rows[b][c % nb], g.at[cdst[b].at[c]],
                        ssem[b][c % nb]).wait()

    return body(inp_flat, t0, t1, t2p, t3p)


def _tc_project(inp_flat, g0, g1, g2, g3, p0t, p1t, p2t, p3t):
    nblk = _TOK // _BT

    def body(x_ref, g0r, g1r, g2r, g3r, p0r, p1r, p2r, p3r, out_ref):
        x = x_ref[...]
        masks = [
            (x >= _CUTS[b]) & (x < _CUTS[b + 1]) for b in range(4)
        ]
        gv0 = jnp.where(masks[0], g0r[...], 0.0).astype(jnp.bfloat16)
        acc = jnp.dot(gv0, p0r[...].astype(jnp.bfloat16),
                      preferred_element_type=jnp.float32)
        gv1 = jnp.where(masks[1], g1r[...], 0.0).astype(jnp.bfloat16)
        acc = acc + jnp.dot(gv1, p1r[...].astype(jnp.bfloat16),
                            preferred_element_type=jnp.float32)
        col = lax.broadcasted_iota(jnp.int32, (_BT, 128), 1)
        for b, gr, pr, colshift in ((2, g2r, p2r, 5), (3, g3r, p3r, 3)):
            pk = _PACK[b]
            sub = (
                jnp.minimum(jnp.maximum(x - _CUTS[b], 0), _SIZES[b] - 1)
                & (pk - 1)
            )
            subm = jnp.where(masks[b], sub, pk)
            cond = lax.shift_right_logical(col, colshift) == subm
            gz = jnp.where(cond, gr[...], 0.0).astype(jnp.bfloat16)
            acc = acc + jnp.dot(gz, pr[...].astype(jnp.bfloat16),
                                preferred_element_type=jnp.float32)
        out_ref[...] = acc * _SCALE

    grid = (nblk,)
    in_specs = [
        pl.BlockSpec((_BT, 1), lambda i: (i, 0)),
        pl.BlockSpec((_BT, _GDIM[0]), lambda i: (i, 0)),
        pl.BlockSpec((_BT, _GDIM[1]), lambda i: (i, 0)),
        pl.BlockSpec((_BT, _GDIM[2]), lambda i: (i, 0)),
        pl.BlockSpec((_BT, _GDIM[3]), lambda i: (i, 0)),
        pl.BlockSpec(p0t.shape, lambda i: (0, 0)),
        pl.BlockSpec(p1t.shape, lambda i: (0, 0)),
        pl.BlockSpec(p2t.shape, lambda i: (0, 0)),
        pl.BlockSpec(p3t.shape, lambda i: (0, 0)),
    ]
    return pl.pallas_call(
        body,
        grid=grid,
        in_specs=in_specs,
        out_specs=pl.BlockSpec((_BT, _D_PROJ), lambda i: (i, 0)),
        out_shape=jax.ShapeDtypeStruct((_TOK, _D_PROJ), jnp.float32),
    )(inp_flat.reshape(_TOK, 1), g0, g1, g2, g3, p0t, p1t, p2t, p3t)


def kernel(inp, table0, proj0, table1, proj1, table2, proj2, table3, proj3):
    inp_flat = inp.reshape(-1)
    t2p = table2.reshape(_SIZES[2] // _PACK[2], 128)
    t3p = table3.reshape(_SIZES[3] // _PACK[3], 128)
    g0, g1, g2, g3 = _sc_gather(inp_flat, table0, table1, t2p, t3p)
    out_flat = _tc_project(
        inp_flat, g0, g1, g2, g3,
        proj0.T, proj1.T,
        jnp.tile(proj2.T, (_PACK[2], 1)), jnp.tile(proj3.T, (_PACK[3], 1)),
    )
    return out_flat.reshape(inp.shape + (_D_PROJ,))


# speedup vs baseline: 1.0041x; 1.0041x over previous
import functools

import jax
import jax.numpy as jnp
from jax import lax
from jax.experimental import pallas as pl
from jax.experimental.pallas import tpu as pltpu
from jax.experimental.pallas import tpu_sc as plsc

_CUTS = (0, 50000, 100000, 180000, 200000)
_SIZES = (50000, 50000, 80000, 20000)
_D_PROJ = 512
_SCALE = float(_D_PROJ) ** 0.5

_GDIM = (512, 128, 128, 128)
_PACK = (1, 1, 4, 16)
_SHIFTS = (0, 0, 2, 4)

_NC, _NS = 2, 16
_NW = _NC * _NS
_TOK = 4 * 8192
_TPW = _TOK // _NW
_CH = 64
_CHB = 6
_NCH = _TPW // _CH
_GROWS = _TOK + _CH

_BT = 1024


def _sc_gather(inp_flat, t0, t1, t2p, t3p):
    mesh = plsc.VectorSubcoreMesh(core_axis_name="c", subcore_axis_name="s")
    out_type = tuple(
        jax.ShapeDtypeStruct((_GROWS, d), jnp.float32) for d in _GDIM
    )
    scratch = [pltpu.VMEM((_TPW,), jnp.int32)]
    scratch += [pltpu.VMEM((_TPW + 16,), jnp.int32)] * 4
    scratch += [pltpu.VMEM((_TPW + 16,), jnp.int32)] * 4
    scratch += [pltpu.VMEM((_NCH, _CH), jnp.int32)] * 4
    scratch += [pltpu.VMEM((_CH, _GDIM[0]), jnp.float32)]
    for d in _GDIM[1:]:
        scratch += [pltpu.VMEM((_CH, d), jnp.float32)] * 2
    scratch += [pltpu.SemaphoreType.DMA] * 16
    scratch += [pltpu.VMEM((48,), jnp.int32)]

    @functools.partial(
        pl.kernel,
        out_type=out_type,
        mesh=mesh,
        scratch_types=scratch,
    )
    def body(inp_hbm, t0h, t1h, t2h, t3h, g0, g1, g2, g3,
             tok_v, *rest):
        cidx = rest[0:4]
        ctok = rest[4:8]
        cdst = rest[8:12]
        rows = [(rest[12], rest[12]), (rest[13], rest[14]),
                (rest[15], rest[16]), (rest[17], rest[18])]
        gsem = [(rest[19 + 2 * b], rest[20 + 2 * b]) for b in range(4)]
        ssem = [(rest[27 + 2 * b], rest[28 + 2 * b]) for b in range(4)]
        psum = rest[35]
        wid = lax.axis_index("s") * _NC + lax.axis_index("c")
        base = wid * _TPW
        pltpu.sync_copy(inp_hbm.at[pl.ds(base, _TPW)], tok_v)

        zero16 = jnp.zeros((16,), jnp.int32)
        trash16 = zero16 + _TOK
        psum[pl.ds(0, 16)] = zero16

        def prefill(j, carry):
            for b in range(4):
                cidx[b][pl.ds(j * 16, 16)] = zero16
                ctok[b][pl.ds(j * 16, 16)] = trash16
            return carry

        lax.fori_loop(0, (_TPW + 16) // 16, prefill, 0)

        def prefix16(v):
            for sh in (1, 2, 4, 8):
                psum[pl.ds(16, 16)] = v
                v = v + psum[pl.ds(16 - sh, 16)]
            return v

        lane = lax.iota(jnp.int32, 16)
        lane1p = lane + 1

        def compact(j, counts):
            x = tok_v[pl.ds(j * 16, 16)]
            tokpos = base + j * 16 + lane
            new_counts = []
            for b in range(4):
                m = (x >= _CUTS[b]) & (x < _CUTS[b + 1])
                ridx = jnp.minimum(
                    jnp.maximum(x - _CUTS[b], 0), _SIZES[b] - 1)
                if _SHIFTS[b]:
                    ridx = lax.shift_right_logical(ridx, _SHIFTS[b])
                cnt = counts[b]
                mi = jnp.where(m, 1, 0)
                pc = prefix16(mi)
                cv1 = zero16
                cv2 = trash16
                for k in range(16):
                    cond = lane1p == pc[k] * mi[k]
                    cv1 = jnp.where(cond, ridx[k], cv1)
                    cv2 = jnp.where(cond, tokpos[k], cv2)
                cidx[b][pl.ds(cnt, 16)] = cv1
                ctok[b][pl.ds(cnt, 16)] = cv2
                new_counts.append(cnt + pc[15])
            return tuple(new_counts)

        counts = lax.fori_loop(
            0, _TPW // 16, compact,
            (jnp.int32(0), jnp.int32(0), jnp.int32(0), jnp.int32(0)))

        def layout_dst(j, carry):
            for b in range(4):
                cdst[b][j // 4, pl.ds((j % 4) * 16, 16)] = (
                    ctok[b][pl.ds(j * 16, 16)])
            return carry

        for j in range(_TPW // 16):
            layout_dst(j, 0)

        tbls = (t0h, t1h, t2h, t3h)
        gouts = (g0, g1, g2, g3)
        nbufs = (1, 2, 2, 2)
        for b in range(4):
            tbl, g, nb = tbls[b], gouts[b], nbufs[b]
            nch = lax.shift_right_logical(counts[b] + (_CH - 1), _CHB)
            for c in range(_NCH):
                @pl.when(c < nch)
                def _(b=b, c=c, tbl=tbl, g=g, nb=nb):
                    buf = rows[b][c % nb]
                    if c >= nb:
                        pltpu.make_async_copy(
                            buf, g.at[cdst[b].at[c - nb]],
                            ssem[b][c % nb]).wait()
                    pltpu.async_copy(
                        tbl.at[cidx[b].at[pl.ds(c * _CH, _CH)]],
                        buf, gsem[b][c % nb]).wait()
                    pltpu.async_copy(
                        buf, g.at[cdst[b].at[c]], ssem[b][c % nb])
            for c in range(_NCH):
                @pl.when(jnp.logical_and(c < nch, c + nbufs[b] >= nch))
                def _(b=b, c=c, g=g, nb=nb):
                    pltpu.make_async_copy(
                        rows[b][c % nb], g.at[cdst[b].at[c]],
                        ssem[b][c % nb]).wait()

    return body(inp_flat, t0, t1, t2p, t3p)


def _tc_project(inp_flat, g0, g1, g2, g3, p0t, p1t, p2t, p3t):
    nblk = _TOK // _BT

    def body(x_ref, g0r, g1r, g2r, g3r, p0r, p1r, p2r, p3r, out_ref):
        x = x_ref[...]
        masks = [
            (x >= _CUTS[b]) & (x < _CUTS[b + 1]) for b in range(4)
        ]
        gv0 = jnp.where(masks[0], g0r[...], 0.0).astype(jnp.bfloat16)
        acc = jnp.dot(gv0, p0r[...].astype(jnp.bfloat16),
                      preferred_element_type=jnp.float32)
        gv1 = jnp.where(masks[1], g1r[...], 0.0).astype(jnp.bfloat16)
        acc = acc + jnp.dot(gv1, p1r[...].astype(jnp.bfloat16),
                            preferred_element_type=jnp.float32)
        col = lax.broadcasted_iota(jnp.int32, (_BT, 128), 1)
        for b, gr, pr, colshift in ((2, g2r, p2r, 5), (3, g3r, p3r, 3)):
            pk = _PACK[b]
            sub = (
                jnp.minimum(jnp.maximum(x - _CUTS[b], 0), _SIZES[b] - 1)
                & (pk - 1)
            )
            subm = jnp.where(masks[b], sub, pk)
            cond = lax.shift_right_logical(col, colshift) == subm
            gz = jnp.where(cond, gr[...], 0.0).astype(jnp.bfloat16)
            acc = acc + jnp.dot(gz, pr[...].astype(jnp.bfloat16),
                                preferred_element_type=jnp.float32)
        out_ref[...] = acc * _SCALE

    grid = (nblk,)
    in_specs = [
        pl.BlockSpec((_BT, 1), lambda i: (i, 0)),
        pl.BlockSpec((_BT, _GDIM[0]), lambda i: (i, 0)),
        pl.BlockSpec((_BT, _GDIM[1]), lambda i: (i, 0)),
        pl.BlockSpec((_BT, _GDIM[2]), lambda i: (i, 0)),
        pl.BlockSpec((_BT, _GDIM[3]), lambda i: (i, 0)),
        pl.BlockSpec(p0t.shape, lambda i: (0, 0)),
        pl.BlockSpec(p1t.shape, lambda i: (0, 0)),
        pl.BlockSpec(p2t.shape, lambda i: (0, 0)),
        pl.BlockSpec(p3t.shape, lambda i: (0, 0)),
    ]
    return pl.pallas_call(
        body,
        grid=grid,
        in_specs=in_specs,
        out_specs=pl.BlockSpec((_BT, _D_PROJ), lambda i: (i, 0)),
        out_shape=jax.ShapeDtypeStruct((_TOK, _D_PROJ), jnp.float32),
    )(inp_flat.reshape(_TOK, 1), g0, g1, g2, g3, p0t, p1t, p2t, p3t)


def kernel(inp, table0, proj0, table1, proj1, table2, proj2, table3, proj3):
    inp_flat = inp.reshape(-1)
    t2p = table2.reshape(_SIZES[2] // _PACK[2], 128)
    t3p = table3.reshape(_SIZES[3] // _PACK[3], 128)
    g0, g1, g2, g3 = _sc_gather(inp_flat, table0, table1, t2p, t3p)
    out_flat = _tc_project(
        inp_flat, g0, g1, g2, g3,
        proj0.T, proj1.T,
        jnp.tile(proj2.T, (_PACK[2], 1)), jnp.tile(proj3.T, (_PACK[3], 1)),
    )
    return out_flat.reshape(inp.shape + (_D_PROJ,))
